# baseline (speedup 1.0000x reference)
import math

import jax
import jax.numpy as jnp
from jax.experimental import pallas as pl
from jax.experimental.pallas import tpu as pltpu

EPS = 0.01
OBS_DIM = 2048
FEAT_DIM = 256
H1 = 1024
H2 = 1024
K = 16
B = 4096
BLK_T = 1024
BLK_H = 256
COLS = 2 * FEAT_DIM + 1
OUT_DIM = K * COLS
LOG_2PI = math.log(2.0 * math.pi)


def _trunk_block(obs_ref, w1_ref, b1_ref, w2_ref, b2_ref, h2_ref):
    f32 = jnp.float32
    h = jnp.maximum(jnp.dot(obs_ref[...], w1_ref[...],
                            preferred_element_type=f32) + b1_ref[...], 0.0)
    h2_ref[...] = jnp.maximum(jnp.dot(h, w2_ref[...],
                                      preferred_element_type=f32) + b2_ref[...], 0.0)


def _heads_block(h2_ref, eps_ref, u_ref, w3_ref, b3_ref,
                 act_ref, ent_ref, mean_ref, w3w_s, b3w_s):
    f32 = jnp.float32

    @pl.when(pl.program_id(0) == 0)
    def _build_logw_weights():
        w3w_s[...] = jnp.concatenate(
            [w3_ref[:, k * COLS:k * COLS + 1] for k in range(K)], axis=1)
        b3w_s[...] = jnp.concatenate(
            [b3_ref[:, k * COLS:k * COLS + 1] for k in range(K)], axis=1)

    h = h2_ref[...]
    out = jnp.dot(h, w3_ref[...], preferred_element_type=f32) + b3_ref[...]
    logw = jnp.dot(h, w3w_s[...], preferred_element_type=f32) + b3w_s[...]

    mus = []
    lss = []
    for k in range(K):
        base = k * COLS
        mus.append(out[:, base + 1:base + 1 + FEAT_DIM])
        lss.append(jnp.clip(out[:, base + 1 + FEAT_DIM:base + COLS], -5.0, 2.0))

    rowmax = jnp.max(logw, axis=1, keepdims=True)
    shifted = logw - rowmax
    log_ws = shifted - jnp.log(jnp.sum(jnp.exp(shifted), axis=1, keepdims=True))

    gumbel = -jnp.log(-jnp.log(u_ref[...]))
    score = log_ws + gumbel
    smax = jnp.max(score, axis=1, keepdims=True)

    found = jnp.zeros_like(smax)
    mu_z = jnp.zeros_like(eps_ref[...])
    ls_z = jnp.zeros_like(eps_ref[...])
    for k in range(K):
        hit = jnp.where((score[:, k:k + 1] >= smax) & (found == 0.0), 1.0, 0.0)
        found = found + hit
        mu_z = mu_z + hit * mus[k]
        ls_z = ls_z + hit * lss[k]

    x = mu_z + jnp.exp(ls_z) * eps_ref[...]

    lp = []
    mean = jnp.zeros_like(x)
    for k in range(K):
        diff = (x - mus[k]) * jnp.exp(-lss[k])
        sumd = jnp.sum(-0.5 * diff * diff - lss[k], axis=1, keepdims=True)
        lp.append(log_ws[:, k:k + 1] + sumd)
        mean = mean + jnp.exp(log_ws[:, k:k + 1]) * mus[k]
    lpmax = lp[0]
    for k in range(1, K):
        lpmax = jnp.maximum(lpmax, lp[k])
    acc = jnp.zeros_like(lpmax)
    for k in range(K):
        acc = acc + jnp.exp(lp[k] - lpmax)
    log_p_x = lpmax + jnp.log(acc) - 0.5 * FEAT_DIM * LOG_2PI

    act = jnp.tanh(x)
    t2 = jnp.tanh(act)
    corr = jnp.sum(jnp.log(1.0 - t2 * t2 + EPS), axis=1, keepdims=True)

    act_ref[...] = act
    ent_ref[...] = -(log_p_x - corr)
    mean_ref[...] = jnp.tanh(mean)


def kernel(obs, eps, u, W1, b1, W2, b2, W3, b3):
    f32 = jnp.float32
    b1r = b1.reshape(1, H1)
    b2r = b2.reshape(1, H2)
    b3r = b3.reshape(1, OUT_DIM)

    row = lambda i: (i, 0)
    const = lambda i: (0, 0)

    h2 = pl.pallas_call(
        _trunk_block,
        grid=(B // BLK_T,),
        in_specs=[
            pl.BlockSpec((BLK_T, OBS_DIM), row),
            pl.BlockSpec((OBS_DIM, H1), const),
            pl.BlockSpec((1, H1), const),
            pl.BlockSpec((H1, H2), const),
            pl.BlockSpec((1, H2), const),
        ],
        out_specs=pl.BlockSpec((BLK_T, H2), row),
        out_shape=jax.ShapeDtypeStruct((B, H2), f32),
        compiler_params=pltpu.CompilerParams(
            dimension_semantics=("arbitrary",),
            vmem_limit_bytes=100 * 1024 * 1024,
        ),
    )(obs, W1, b1r, W2, b2r)

    act, ent, mean = pl.pallas_call(
        _heads_block,
        grid=(B // BLK_H,),
        in_specs=[
            pl.BlockSpec((BLK_H, H2), row),
            pl.BlockSpec((BLK_H, FEAT_DIM), row),
            pl.BlockSpec((BLK_H, K), row),
            pl.BlockSpec((H2, OUT_DIM), const),
            pl.BlockSpec((1, OUT_DIM), const),
        ],
        out_specs=[
            pl.BlockSpec((BLK_H, FEAT_DIM), row),
            pl.BlockSpec((BLK_H, 1), row),
            pl.BlockSpec((BLK_H, FEAT_DIM), row),
        ],
        out_shape=[
            jax.ShapeDtypeStruct((B, FEAT_DIM), f32),
            jax.ShapeDtypeStruct((B, 1), f32),
            jax.ShapeDtypeStruct((B, FEAT_DIM), f32),
        ],
        scratch_shapes=[
            pltpu.VMEM((H2, K), f32),
            pltpu.VMEM((1, K), f32),
        ],
        compiler_params=pltpu.CompilerParams(
            dimension_semantics=("arbitrary",),
            vmem_limit_bytes=100 * 1024 * 1024,
        ),
    )(h2, eps, u, W3, b3r)
    return act, ent, mean


# speedup vs baseline: 1.3560x; 1.3560x over previous
import math

import jax
import jax.numpy as jnp
from jax.experimental import pallas as pl
from jax.experimental.pallas import tpu as pltpu

EPS = 0.01
OBS_DIM = 2048
FEAT_DIM = 256
H1 = 1024
H2 = 1024
K = 16
B = 4096
BLK_T = 1024
BLK_H = 256
COLS = 2 * FEAT_DIM + 1
OUT_DIM = K * COLS
LOG_2PI = math.log(2.0 * math.pi)


def _trunk_block(obs_ref, w1_ref, b1_ref, w2_ref, b2_ref, h2_ref):
    f32 = jnp.float32
    h = jnp.maximum(jnp.dot(obs_ref[...], w1_ref[...],
                            preferred_element_type=f32) + b1_ref[...], 0.0)
    h2_ref[...] = jnp.maximum(jnp.dot(h, w2_ref[...],
                                      preferred_element_type=f32) + b2_ref[...], 0.0)


def _heads_block(h2_ref, eps_ref, u_ref, w3ms_ref, b3ms_ref, w3w_ref, b3w_ref,
                 act_ref, ent_ref, mean_ref):
    f32 = jnp.float32
    h = h2_ref[...]
    out = jnp.dot(h, w3ms_ref[...], preferred_element_type=f32) + b3ms_ref[...]
    logw = jnp.dot(h, w3w_ref[...], preferred_element_type=f32) + b3w_ref[...]

    mus = []
    lss = []
    for k in range(K):
        base = 2 * k * FEAT_DIM
        mus.append(out[:, base:base + FEAT_DIM])
        lss.append(jnp.clip(out[:, base + FEAT_DIM:base + 2 * FEAT_DIM], -5.0, 2.0))

    rowmax = jnp.max(logw, axis=1, keepdims=True)
    shifted = logw - rowmax
    log_ws = shifted - jnp.log(jnp.sum(jnp.exp(shifted), axis=1, keepdims=True))

    gumbel = -jnp.log(-jnp.log(u_ref[...]))
    score = log_ws + gumbel
    smax = jnp.max(score, axis=1, keepdims=True)

    found = jnp.zeros_like(smax)
    mu_z = jnp.zeros_like(eps_ref[...])
    ls_z = jnp.zeros_like(eps_ref[...])
    for k in range(K):
        hit = jnp.where((score[:, k:k + 1] >= smax) & (found == 0.0), 1.0, 0.0)
        found = found + hit
        mu_z = mu_z + hit * mus[k]
        ls_z = ls_z + hit * lss[k]

    x = mu_z + jnp.exp(ls_z) * eps_ref[...]

    lp = []
    mean = jnp.zeros_like(x)
    for k in range(K):
        diff = (x - mus[k]) * jnp.exp(-lss[k])
        sumd = jnp.sum(-0.5 * diff * diff - lss[k], axis=1, keepdims=True)
        lp.append(log_ws[:, k:k + 1] + sumd)
        mean = mean + jnp.exp(log_ws[:, k:k + 1]) * mus[k]
    lpmax = lp[0]
    for k in range(1, K):
        lpmax = jnp.maximum(lpmax, lp[k])
    acc = jnp.zeros_like(lpmax)
    for k in range(K):
        acc = acc + jnp.exp(lp[k] - lpmax)
    log_p_x = lpmax + jnp.log(acc) - 0.5 * FEAT_DIM * LOG_2PI

    act = jnp.tanh(x)
    t2 = jnp.tanh(act)
    corr = jnp.sum(jnp.log(1.0 - t2 * t2 + EPS), axis=1, keepdims=True)

    act_ref[...] = act
    ent_ref[...] = -(log_p_x - corr)
    mean_ref[...] = jnp.tanh(mean)


def kernel(obs, eps, u, W1, b1, W2, b2, W3, b3):
    f32 = jnp.float32
    b1r = b1.reshape(1, H1)
    b2r = b2.reshape(1, H2)
    W3r = W3.reshape(H2, K, COLS)
    W3ms = jax.lax.slice(W3r, (0, 0, 1), (H2, K, COLS)).reshape(H2, K * 2 * FEAT_DIM)
    W3w = jax.lax.slice(W3r, (0, 0, 0), (H2, K, 1)).reshape(H2, K)
    b3r3 = b3.reshape(1, K, COLS)
    b3ms = jax.lax.slice(b3r3, (0, 0, 1), (1, K, COLS)).reshape(1, K * 2 * FEAT_DIM)
    b3w = jax.lax.slice(b3r3, (0, 0, 0), (1, K, 1)).reshape(1, K)

    row = lambda i: (i, 0)
    const = lambda i: (0, 0)

    h2 = pl.pallas_call(
        _trunk_block,
        grid=(B // BLK_T,),
        in_specs=[
            pl.BlockSpec((BLK_T, OBS_DIM), row),
            pl.BlockSpec((OBS_DIM, H1), const),
            pl.BlockSpec((1, H1), const),
            pl.BlockSpec((H1, H2), const),
            pl.BlockSpec((1, H2), const),
        ],
        out_specs=pl.BlockSpec((BLK_T, H2), row),
        out_shape=jax.ShapeDtypeStruct((B, H2), f32),
        compiler_params=pltpu.CompilerParams(
            dimension_semantics=("arbitrary",),
            vmem_limit_bytes=100 * 1024 * 1024,
        ),
    )(obs, W1, b1r, W2, b2r)

    act, ent, mean = pl.pallas_call(
        _heads_block,
        grid=(B // BLK_H,),
        in_specs=[
            pl.BlockSpec((BLK_H, H2), row),
            pl.BlockSpec((BLK_H, FEAT_DIM), row),
            pl.BlockSpec((BLK_H, K), row),
            pl.BlockSpec((H2, K * 2 * FEAT_DIM), const),
            pl.BlockSpec((1, K * 2 * FEAT_DIM), const),
            pl.BlockSpec((H2, K), const),
            pl.BlockSpec((1, K), const),
        ],
        out_specs=[
            pl.BlockSpec((BLK_H, FEAT_DIM), row),
            pl.BlockSpec((BLK_H, 1), row),
            pl.BlockSpec((BLK_H, FEAT_DIM), row),
        ],
        out_shape=[
            jax.ShapeDtypeStruct((B, FEAT_DIM), f32),
            jax.ShapeDtypeStruct((B, 1), f32),
            jax.ShapeDtypeStruct((B, FEAT_DIM), f32),
        ],
        compiler_params=pltpu.CompilerParams(
            dimension_semantics=("arbitrary",),
            vmem_limit_bytes=100 * 1024 * 1024,
        ),
    )(h2, eps, u, W3ms, b3ms, W3w, b3w)
    return act, ent, mean


# speedup vs baseline: 1.5630x; 1.1527x over previous
import math

import jax
import jax.numpy as jnp
from jax.experimental import pallas as pl
from jax.experimental.pallas import tpu as pltpu

EPS = 0.01
OBS_DIM = 2048
FEAT_DIM = 256
H1 = 1024
H2 = 1024
K = 16
B = 4096
BLK = 256
COLS = 2 * FEAT_DIM + 1
LOG_2PI = math.log(2.0 * math.pi)

def _gmm_block(obs_ref, eps_ref, u_ref, w1_ref, b1_ref, w2_ref, b2_ref,
               w3w_ref, b3w_ref, w3mu_ref, b3mu_ref, w3sig_ref, b3sig_ref,
               ones_ref, act_ref, ent_ref, mean_ref):
    f32 = jnp.float32
    h = jnp.maximum(jnp.dot(obs_ref[...], w1_ref[...],
                            preferred_element_type=f32) + b1_ref[...], 0.0)
    h = jnp.maximum(jnp.dot(h, w2_ref[...],
                            preferred_element_type=f32) + b2_ref[...], 0.0)
    logw = jnp.dot(h, w3w_ref[...], preferred_element_type=f32) + b3w_ref[...]
    mu_all = jnp.dot(h, w3mu_ref[...], preferred_element_type=f32) + b3mu_ref[...]
    ls_all = jnp.clip(
        jnp.dot(h, w3sig_ref[...], preferred_element_type=f32) + b3sig_ref[...],
        -5.0, 2.0)

    rowmax = jnp.max(logw, axis=1, keepdims=True)
    shifted = logw - rowmax
    log_ws = shifted - jnp.log(jnp.sum(jnp.exp(shifted), axis=1, keepdims=True))

    gumbel = -jnp.log(-jnp.log(u_ref[...]))
    score = log_ws + gumbel
    smax = jnp.max(score, axis=1, keepdims=True)

    found = jnp.zeros_like(smax)
    mu_z = jnp.zeros_like(eps_ref[...])
    ls_z = jnp.zeros_like(eps_ref[...])
    for k in range(K):
        hit = jnp.where((score[:, k:k + 1] >= smax) & (found == 0.0), 1.0, 0.0)
        found = found + hit
        sl = slice(k * FEAT_DIM, (k + 1) * FEAT_DIM)
        mu_z = mu_z + hit * mu_all[:, sl]
        ls_z = ls_z + hit * ls_all[:, sl]

    x = mu_z + jnp.exp(ls_z) * eps_ref[...]

    w_mat = jnp.exp(log_ws)
    lps = []
    mean = jnp.zeros_like(x)
    for k in range(K):
        sl = slice(k * FEAT_DIM, (k + 1) * FEAT_DIM)
        ls_k = ls_all[:, sl]
        mu_k = mu_all[:, sl]
        diff = (x - mu_k) * jnp.exp(-ls_k)
        p_k = -0.5 * diff * diff - ls_k
        sumd = jnp.dot(p_k, ones_ref[...], preferred_element_type=f32)
        lps.append(log_ws[:, k:k + 1] + sumd)
        mean = mean + w_mat[:, k:k + 1] * mu_k
    lpmax = lps[0]
    for k in range(1, K):
        lpmax = jnp.maximum(lpmax, lps[k])
    acc = jnp.zeros_like(lpmax)
    for k in range(K):
        acc = acc + jnp.exp(lps[k] - lpmax)
    log_p_x = lpmax + jnp.log(acc) - 0.5 * FEAT_DIM * LOG_2PI

    act = jnp.tanh(x)
    t2 = jnp.tanh(act)
    corr = jnp.dot(jnp.log(1.0 - t2 * t2 + EPS), ones_ref[...],
                   preferred_element_type=f32)

    act_ref[...] = act
    ent_ref[...] = -(log_p_x - corr)
    mean_ref[...] = jnp.tanh(mean)


def kernel(obs, eps, u, W1, b1, W2, b2, W3, b3):
    f32 = jnp.float32
    W3r = W3.reshape(H2, K, COLS)
    W3w = W3r[:, :, 0]
    W3mu = W3r[:, :, 1:1 + FEAT_DIM].reshape(H2, K * FEAT_DIM)
    W3sig = W3r[:, :, 1 + FEAT_DIM:].reshape(H2, K * FEAT_DIM)
    b3r = b3.reshape(K, COLS)
    b3w = b3r[:, 0].reshape(1, K)
    b3mu = b3r[:, 1:1 + FEAT_DIM].reshape(1, K * FEAT_DIM)
    b3sig = b3r[:, 1 + FEAT_DIM:].reshape(1, K * FEAT_DIM)
    b1r = b1.reshape(1, H1)
    b2r = b2.reshape(1, H2)
    ones_col = jnp.ones((FEAT_DIM, 1), f32)

    nblk = B // BLK
    row = lambda i: (i, 0)
    const = lambda i: (0, 0)

    act, ent, mean = pl.pallas_call(
        _gmm_block,
        grid=(nblk,),
        in_specs=[
            pl.BlockSpec((BLK, OBS_DIM), row),
            pl.BlockSpec((BLK, FEAT_DIM), row),
            pl.BlockSpec((BLK, K), row),
            pl.BlockSpec((OBS_DIM, H1), const),
            pl.BlockSpec((1, H1), const),
            pl.BlockSpec((H1, H2), const),
            pl.BlockSpec((1, H2), const),
            pl.BlockSpec((H2, K), const),
            pl.BlockSpec((1, K), const),
            pl.BlockSpec((H2, K * FEAT_DIM), const),
            pl.BlockSpec((1, K * FEAT_DIM), const),
            pl.BlockSpec((H2, K * FEAT_DIM), const),
            pl.BlockSpec((1, K * FEAT_DIM), const),
            pl.BlockSpec((FEAT_DIM, 1), const),
        ],
        out_specs=[
            pl.BlockSpec((BLK, FEAT_DIM), row),
            pl.BlockSpec((BLK, 1), row),
            pl.BlockSpec((BLK, FEAT_DIM), row),
        ],
        out_shape=[
            jax.ShapeDtypeStruct((B, FEAT_DIM), f32),
            jax.ShapeDtypeStruct((B, 1), f32),
            jax.ShapeDtypeStruct((B, FEAT_DIM), f32),
        ],
        compiler_params=pltpu.CompilerParams(
            dimension_semantics=("arbitrary",),
            vmem_limit_bytes=100 * 1024 * 1024,
        ),
    )(obs, eps, u, W1, b1r, W2, b2r, W3w, b3w, W3mu, b3mu, W3sig, b3sig, ones_col)
    return act, ent, mean


# speedup vs baseline: 1.5853x; 1.0143x over previous
import math

import jax
import jax.numpy as jnp
from jax.experimental import pallas as pl
from jax.experimental.pallas import tpu as pltpu

EPS = 0.01
OBS_DIM = 2048
FEAT_DIM = 256
H1 = 1024
H2 = 1024
K = 16
B = 4096
BLK = 256
COLS = 2 * FEAT_DIM + 1
LOG_2PI = math.log(2.0 * math.pi)

def _gmm_block(obs_ref, eps_ref, u_ref, w1_ref, b1_ref, w2_ref, b2_ref,
               w3w_ref, b3w_ref, w3mu_ref, b3mu_ref, w3sig_ref, b3sig_ref,
               act_ref, ent_ref, mean_ref):
    f32 = jnp.float32
    h = jnp.maximum(jnp.dot(obs_ref[...], w1_ref[...],
                            preferred_element_type=f32) + b1_ref[...], 0.0)
    h = jnp.maximum(jnp.dot(h, w2_ref[...],
                            preferred_element_type=f32) + b2_ref[...], 0.0)
    logw = jnp.dot(h, w3w_ref[...], preferred_element_type=f32) + b3w_ref[...]
    mu_all = jnp.dot(h, w3mu_ref[...], preferred_element_type=f32) + b3mu_ref[...]
    ls_all = jnp.clip(
        jnp.dot(h, w3sig_ref[...], preferred_element_type=f32) + b3sig_ref[...],
        -5.0, 2.0)

    rowmax = jnp.max(logw, axis=1, keepdims=True)
    shifted = logw - rowmax
    log_ws = shifted - jnp.log(jnp.sum(jnp.exp(shifted), axis=1, keepdims=True))

    gumbel = -jnp.log(-jnp.log(u_ref[...]))
    score = log_ws + gumbel
    smax = jnp.max(score, axis=1, keepdims=True)

    found = jnp.zeros_like(smax)
    mu_z = jnp.zeros_like(eps_ref[...])
    ls_z = jnp.zeros_like(eps_ref[...])
    for k in range(K):
        hit = jnp.where((score[:, k:k + 1] >= smax) & (found == 0.0), 1.0, 0.0)
        found = found + hit
        sl = slice(k * FEAT_DIM, (k + 1) * FEAT_DIM)
        mu_z = mu_z + hit * mu_all[:, sl]
        ls_z = ls_z + hit * ls_all[:, sl]

    x = mu_z + jnp.exp(ls_z) * eps_ref[...]

    w_mat = jnp.exp(log_ws)
    lps = []
    mean = jnp.zeros_like(x)
    for k in range(K):
        sl = slice(k * FEAT_DIM, (k + 1) * FEAT_DIM)
        ls_k = ls_all[:, sl]
        mu_k = mu_all[:, sl]
        diff = (x - mu_k) * jnp.exp(-ls_k)
        p_k = -0.5 * diff * diff - ls_k
        sumd = jnp.sum(p_k, axis=1, keepdims=True)
        lps.append(log_ws[:, k:k + 1] + sumd)
        mean = mean + w_mat[:, k:k + 1] * mu_k
    lpmax = lps[0]
    for k in range(1, K):
        lpmax = jnp.maximum(lpmax, lps[k])
    acc = jnp.zeros_like(lpmax)
    for k in range(K):
        acc = acc + jnp.exp(lps[k] - lpmax)
    log_p_x = lpmax + jnp.log(acc) - 0.5 * FEAT_DIM * LOG_2PI

    act = jnp.tanh(x)
    t2 = jnp.tanh(act)
    corr = jnp.sum(jnp.log(1.0 - t2 * t2 + EPS), axis=1, keepdims=True)

    act_ref[...] = act
    ent_ref[...] = -(log_p_x - corr)
    mean_ref[...] = jnp.tanh(mean)


def kernel(obs, eps, u, W1, b1, W2, b2, W3, b3):
    f32 = jnp.float32
    W3r = W3.reshape(H2, K, COLS)
    W3w = W3r[:, :, 0]
    W3mu = W3r[:, :, 1:1 + FEAT_DIM].reshape(H2, K * FEAT_DIM)
    W3sig = W3r[:, :, 1 + FEAT_DIM:].reshape(H2, K * FEAT_DIM)
    b3r = b3.reshape(K, COLS)
    b3w = b3r[:, 0].reshape(1, K)
    b3mu = b3r[:, 1:1 + FEAT_DIM].reshape(1, K * FEAT_DIM)
    b3sig = b3r[:, 1 + FEAT_DIM:].reshape(1, K * FEAT_DIM)
    b1r = b1.reshape(1, H1)
    b2r = b2.reshape(1, H2)

    nblk = B // BLK
    row = lambda i: (i, 0)
    const = lambda i: (0, 0)

    act, ent, mean = pl.pallas_call(
        _gmm_block,
        grid=(nblk,),
        in_specs=[
            pl.BlockSpec((BLK, OBS_DIM), row),
            pl.BlockSpec((BLK, FEAT_DIM), row),
            pl.BlockSpec((BLK, K), row),
            pl.BlockSpec((OBS_DIM, H1), const),
            pl.BlockSpec((1, H1), const),
            pl.BlockSpec((H1, H2), const),
            pl.BlockSpec((1, H2), const),
            pl.BlockSpec((H2, K), const),
            pl.BlockSpec((1, K), const),
            pl.BlockSpec((H2, K * FEAT_DIM), const),
            pl.BlockSpec((1, K * FEAT_DIM), const),
            pl.BlockSpec((H2, K * FEAT_DIM), const),
            pl.BlockSpec((1, K * FEAT_DIM), const),
        ],
        out_specs=[
            pl.BlockSpec((BLK, FEAT_DIM), row),
            pl.BlockSpec((BLK, 1), row),
            pl.BlockSpec((BLK, FEAT_DIM), row),
        ],
        out_shape=[
            jax.ShapeDtypeStruct((B, FEAT_DIM), f32),
            jax.ShapeDtypeStruct((B, 1), f32),
            jax.ShapeDtypeStruct((B, FEAT_DIM), f32),
        ],
        compiler_params=pltpu.CompilerParams(
            dimension_semantics=("arbitrary",),
            vmem_limit_bytes=100 * 1024 * 1024,
        ),
    )(obs, eps, u, W1, b1r, W2, b2r, W3w, b3w, W3mu, b3mu, W3sig, b3sig)
    return act, ent, mean
